# Initial kernel scaffold; baseline (speedup 1.0000x reference)
import jax
import jax.numpy as jnp
from jax.experimental import pallas as pl


def kernel(x, edge_index, edge_weight, batch, W_z, b_z, W_r, b_r, W_h, b_h, W_lin, b_lin):
    raise NotImplementedError("write your pallas kernel here")


# speedup vs baseline: 3.2393x; 3.2393x over previous
import functools

import jax
import jax.numpy as jnp
from jax import lax
from jax.experimental import pallas as pl
from jax.experimental.pallas import tpu as pltpu
from jax.experimental.pallas import tpu_sc as plsc

N = 10000
E = 320000
F = 128
HID = 32
KHOP = 5
G = 16
NCLS = 10
W64 = 2 * HID
FLAT = 2 * N
NC, NS = 2, 16
NW = NC * NS
EPW = E // NW
NPAD = NS * 640
CH = 6400
NCH = E // CH

_f32 = jnp.float32
_i32 = jnp.int32


def _mesh():
    return plsc.VectorSubcoreMesh(core_axis_name="c", subcore_axis_name="s",
                                  num_cores=NC, num_subcores=NS)


def _wid():
    return lax.axis_index("s") * NC + lax.axis_index("c")


def _deg_body(packed, ew, out_o, out_i, loc_o, loc_i, ebuf, wbuf, acc, tmp,
              sh_o, sh_i):
    c = lax.axis_index("c")
    s = lax.axis_index("s")
    wid = s * NC + c

    def zero(i, _):
        loc_o[pl.ds(i * 16, 16)] = jnp.zeros((16,), _f32)
        loc_i[pl.ds(i * 16, 16)] = jnp.zeros((16,), _f32)
        return 0

    lax.fori_loop(0, NPAD // 16, zero, 0)

    base = wid * EPW
    pltpu.sync_copy(packed.at[pl.ds(base, EPW)], ebuf)
    pltpu.sync_copy(ew.at[pl.ds(base, EPW)], wbuf)

    def blk(i, _):
        p = ebuf[pl.ds(i * 16, 16)]
        w = wbuf[pl.ds(i * 16, 16)]
        r = p & 0xFFFF
        cc = lax.shift_right_logical(p, 16)
        plsc.addupdate_scatter(loc_o, [r], w)
        plsc.addupdate_scatter(loc_i, [cc], w)
        return 0

    lax.fori_loop(0, EPW // 16, blk, 0)

    pltpu.sync_copy(loc_o, sh_o.at[s])
    pltpu.sync_copy(loc_i, sh_i.at[s])
    plsc.subcore_barrier()

    off = s * 640
    for sh, outref in ((sh_o, out_o), (sh_i, out_i)):
        def zero2(i, _):
            acc[pl.ds(i * 16, 16)] = jnp.zeros((16,), _f32)
            return 0

        lax.fori_loop(0, 40, zero2, 0)
        for r in range(NS):
            pltpu.sync_copy(sh.at[r, pl.ds(off, 640)], tmp)

            def addl(i, _):
                acc[pl.ds(i * 16, 16)] = (acc[pl.ds(i * 16, 16)]
                                          + tmp[pl.ds(i * 16, 16)])
                return 0

            lax.fori_loop(0, 40, addl, 0)
        pltpu.sync_copy(acc, outref.at[pl.ds(c * NPAD + off, 640)])


_deg_kernel = functools.partial(
    pl.kernel,
    out_type=(jax.ShapeDtypeStruct((NC * NPAD,), _f32),
              jax.ShapeDtypeStruct((NC * NPAD,), _f32)),
    mesh=_mesh(),
    compiler_params=pltpu.CompilerParams(needs_layout_passes=False),
    scratch_types=[
        pltpu.VMEM((NPAD,), _f32), pltpu.VMEM((NPAD,), _f32),
        pltpu.VMEM((EPW,), _i32), pltpu.VMEM((EPW,), _f32),
        pltpu.VMEM((640,), _f32), pltpu.VMEM((640,), _f32),
        pltpu.VMEM_SHARED((NS, NPAD), _f32),
        pltpu.VMEM_SHARED((NS, NPAD), _f32),
    ],
)(_deg_body)


def _coef_body(po, pi, packed, ew, coefo, coefi,
               ideg_o, ideg_i, tmp, ebuf, wbuf, cob, cib):
    wid = _wid()

    for pref, idst in ((po, ideg_o), (pi, ideg_i)):
        pltpu.sync_copy(pref.at[pl.ds(0, N)], idst)
        pltpu.sync_copy(pref.at[pl.ds(NPAD, N)], tmp)

        def inv(i, _):
            v = idst[pl.ds(i * 16, 16)] + tmp[pl.ds(i * 16, 16)]
            idst[pl.ds(i * 16, 16)] = jnp.where(
                v > 0.0, 1.0 / v, jnp.zeros((16,), _f32))
            return 0

        lax.fori_loop(0, N // 16, inv, 0)

    base = wid * EPW
    pltpu.sync_copy(packed.at[pl.ds(base, EPW)], ebuf)
    pltpu.sync_copy(ew.at[pl.ds(base, EPW)], wbuf)

    def blk(i, _):
        p = ebuf[pl.ds(i * 16, 16)]
        w = wbuf[pl.ds(i * 16, 16)]
        r = p & 0xFFFF
        cc = lax.shift_right_logical(p, 16)
        cob[pl.ds(i * 16, 16)] = plsc.load_gather(ideg_o, [r]) * w
        cib[pl.ds(i * 16, 16)] = plsc.load_gather(ideg_i, [cc]) * w
        return 0

    lax.fori_loop(0, EPW // 16, blk, 0)
    pltpu.sync_copy(cob, coefo.at[pl.ds(base, EPW)])
    pltpu.sync_copy(cib, coefi.at[pl.ds(base, EPW)])


_coef_kernel = functools.partial(
    pl.kernel,
    out_type=(jax.ShapeDtypeStruct((E,), _f32),
              jax.ShapeDtypeStruct((E,), _f32)),
    mesh=_mesh(),
    compiler_params=pltpu.CompilerParams(needs_layout_passes=False),
    scratch_types=[
        pltpu.VMEM((N,), _f32), pltpu.VMEM((N,), _f32), pltpu.VMEM((N,), _f32),
        pltpu.VMEM((EPW,), _i32), pltpu.VMEM((EPW,), _f32),
        pltpu.VMEM((EPW,), _f32), pltpu.VMEM((EPW,), _f32),
    ],
)(_coef_body)


def _clen_body(Yw, packed, coefo, coefi, out, B0, B1, B2, S, ebuf, cbuf):
    wid = _wid()

    def ew_sub(dst, sub):
        def body(i, _):
            sl = pl.ds(i * 16, 16)
            dst[sl] = dst[sl] - sub[sl]
            return 0

        lax.fori_loop(0, FLAT // 16, body, 0)

    def prop(srcb, dstb, coef_hbm, fac, src_lo):
        def chunk(ch, _):
            off = pl.multiple_of(ch * CH, 256)
            pltpu.sync_copy(packed.at[pl.ds(off, CH)], ebuf)
            pltpu.sync_copy(coef_hbm.at[pl.ds(off, CH)], cbuf)

            def blk(i, _):
                sl = pl.ds(i * 16, 16)
                p = ebuf[sl]
                cf = cbuf[sl]
                lo = p & 0xFFFF
                hi = lax.shift_right_logical(p, 16)
                si = lo if src_lo else hi
                di = hi if src_lo else lo
                s2 = si + si
                d2 = di + di
                t = cf + cf if fac == 2.0 else cf
                v0 = plsc.load_gather(srcb, [s2])
                v1 = plsc.load_gather(srcb, [s2 + 1])
                plsc.addupdate_scatter(dstb, [d2], v0 * t)
                plsc.addupdate_scatter(dstb, [d2 + 1], v1 * t)
                return 0

            lax.fori_loop(0, CH // 16, blk, 0)
            return 0

        lax.fori_loop(0, NCH, chunk, 0)

    for d, (coef_hbm, src_lo) in enumerate(((coefo, True), (coefi, False))):
        bk1, bnew, free = B0, B1, B2
        pltpu.sync_copy(Yw.at[pl.ds(((d * KHOP + KHOP - 1) * NW + wid) * FLAT, FLAT)], bk1)
        pltpu.sync_copy(Yw.at[pl.ds(((d * KHOP + KHOP - 2) * NW + wid) * FLAT, FLAT)], bnew)
        prop(bk1, bnew, coef_hbm, 2.0, src_lo)
        bk2, bk1 = bk1, bnew
        for k in range(KHOP - 3, 0, -1):
            bnew = free
            pltpu.sync_copy(Yw.at[pl.ds(((d * KHOP + k) * NW + wid) * FLAT, FLAT)], bnew)
            ew_sub(bnew, bk2)
            prop(bk1, bnew, coef_hbm, 2.0, src_lo)
            free, bk2, bk1 = bk2, bk1, bnew
        pltpu.sync_copy(Yw.at[pl.ds((d * KHOP * NW + wid) * FLAT, FLAT)], free)

        def fin(i, _):
            sl = pl.ds(i * 16, 16)
            v = free[sl] - bk2[sl]
            S[sl] = v if d == 0 else S[sl] + v
            return 0

        lax.fori_loop(0, FLAT // 16, fin, 0)
        prop(bk1, S, coef_hbm, 1.0, src_lo)

    pltpu.sync_copy(S, out.at[pl.ds(wid * FLAT, FLAT)])


_clen_kernel = functools.partial(
    pl.kernel,
    out_type=jax.ShapeDtypeStruct((NW * FLAT,), _f32),
    mesh=_mesh(),
    compiler_params=pltpu.CompilerParams(needs_layout_passes=False),
    scratch_types=[
        pltpu.VMEM((FLAT,), _f32), pltpu.VMEM((FLAT,), _f32),
        pltpu.VMEM((FLAT,), _f32), pltpu.VMEM((FLAT,), _f32),
        pltpu.VMEM((CH,), _i32), pltpu.VMEM((CH,), _f32),
    ],
)(_clen_body)


def _mm_body(xr, wr, yr):
    yr[...] = jnp.dot(xr[...], wr[...], preferred_element_type=_f32)


_mm_kernel = pl.pallas_call(
    _mm_body,
    grid=(10,),
    in_specs=[pl.BlockSpec((N // 10, F), lambda i: (i, 0)),
              pl.BlockSpec((F, 2 * KHOP * W64), lambda i: (0, 0))],
    out_specs=pl.BlockSpec((N // 10, 2 * KHOP * W64), lambda i: (i, 0)),
    out_shape=jax.ShapeDtypeStruct((N, 2 * KHOP * W64), _f32),
)


def _fin_body(s_ref, b_ref, bz_ref, bh_ref, wl_ref, bl_ref, out_ref):
    sv = s_ref[...]
    z = jax.nn.sigmoid(sv[:, :HID] + bz_ref[...])
    ht = jnp.tanh(sv[:, HID:] + bh_ref[...])
    hr = jnp.maximum((1.0 - z) * ht, 0.0)
    oh = (b_ref[...] == lax.broadcasted_iota(_i32, (N, G), 1)).astype(_f32)
    sums = lax.dot_general(oh, hr, (((0,), (0,)), ((), ())),
                           preferred_element_type=_f32)
    cnts = lax.dot_general(oh, jnp.ones((N, 1), _f32), (((0,), (0,)), ((), ())),
                           preferred_element_type=_f32)
    pooled = sums / jnp.maximum(cnts, 1.0)
    out_ref[...] = jnp.dot(pooled, wl_ref[...],
                           preferred_element_type=_f32) + bl_ref[...]


_fin_kernel = pl.pallas_call(
    _fin_body,
    out_shape=jax.ShapeDtypeStruct((G, NCLS), _f32),
)


def kernel(x, edge_index, edge_weight, batch, W_z, b_z, W_r, b_r, W_h, b_h,
           W_lin, b_lin):
    row = edge_index[0].astype(_i32)
    col = edge_index[1].astype(_i32)
    packed = row | (col << 16)

    po, pi = _deg_kernel(packed, edge_weight)
    coef_o, coef_i = _coef_kernel(po, pi, packed, edge_weight)

    wcat = jnp.concatenate([W_z[:, :, :F, :], W_h[:, :, :F, :]], axis=-1)
    wflat = wcat.reshape(2 * KHOP, F, W64).transpose(1, 0, 2).reshape(
        F, 2 * KHOP * W64)
    y = _mm_kernel(x, wflat)
    yw = y.reshape(N, 2 * KHOP, NW, 2).transpose(1, 2, 0, 3).reshape(
        2 * KHOP * NW * FLAT)

    sw = _clen_kernel(yw, packed, coef_o, coef_i)
    s = sw.reshape(NW, N, 2).transpose(1, 0, 2).reshape(N, W64)

    return _fin_kernel(s, batch.reshape(N, 1).astype(_i32),
                       b_z.reshape(1, HID), b_h.reshape(1, HID),
                       W_lin, b_lin.reshape(1, NCLS))


# speedup vs baseline: 8.8763x; 2.7402x over previous
import functools

import jax
import jax.numpy as jnp
from jax import lax
from jax.experimental import pallas as pl
from jax.experimental.pallas import tpu as pltpu
from jax.experimental.pallas import tpu_sc as plsc

N = 10000
E = 320000
F = 128
HID = 32
KHOP = 5
G = 16
NCLS = 10
W64 = 2 * HID
NC, NS = 2, 16
NW = NC * NS
EPW = E // NW
NPAD = NS * 640
CH = 12800
NCH = E // CH
NCOL = 2 * KHOP * W64

_f32 = jnp.float32
_i32 = jnp.int32


def _mesh():
    return plsc.VectorSubcoreMesh(core_axis_name="c", subcore_axis_name="s",
                                  num_cores=NC, num_subcores=NS)


def _wid():
    return lax.axis_index("s") * NC + lax.axis_index("c")


def _deg_body(packed, ew, out_o, out_i, loc_o, loc_i, ebuf, wbuf, acc, tmp,
              sh_o, sh_i):
    c = lax.axis_index("c")
    s = lax.axis_index("s")
    wid = s * NC + c

    def zero(i, _):
        loc_o[pl.ds(i * 16, 16)] = jnp.zeros((16,), _f32)
        loc_i[pl.ds(i * 16, 16)] = jnp.zeros((16,), _f32)
        return 0

    lax.fori_loop(0, NPAD // 16, zero, 0)

    base = wid * EPW
    pltpu.sync_copy(packed.at[pl.ds(base, EPW)], ebuf)
    pltpu.sync_copy(ew.at[pl.ds(base, EPW)], wbuf)

    def blk(i, _):
        p = ebuf[pl.ds(i * 16, 16)]
        w = wbuf[pl.ds(i * 16, 16)]
        r = p & 0xFFFF
        cc = lax.shift_right_logical(p, 16)
        plsc.addupdate_scatter(loc_o, [r], w)
        plsc.addupdate_scatter(loc_i, [cc], w)
        return 0

    lax.fori_loop(0, EPW // 16, blk, 0)

    pltpu.sync_copy(loc_o, sh_o.at[s])
    pltpu.sync_copy(loc_i, sh_i.at[s])
    plsc.subcore_barrier()

    off = s * 640
    for sh, outref in ((sh_o, out_o), (sh_i, out_i)):
        def zero2(i, _):
            acc[pl.ds(i * 16, 16)] = jnp.zeros((16,), _f32)
            return 0

        lax.fori_loop(0, 40, zero2, 0)
        for r in range(NS):
            pltpu.sync_copy(sh.at[r, pl.ds(off, 640)], tmp)

            def addl(i, _):
                acc[pl.ds(i * 16, 16)] = (acc[pl.ds(i * 16, 16)]
                                          + tmp[pl.ds(i * 16, 16)])
                return 0

            lax.fori_loop(0, 40, addl, 0)
        pltpu.sync_copy(acc, outref.at[pl.ds(c * NPAD + off, 640)])


_deg_kernel = functools.partial(
    pl.kernel,
    out_type=(jax.ShapeDtypeStruct((NC * NPAD,), _f32),
              jax.ShapeDtypeStruct((NC * NPAD,), _f32)),
    mesh=_mesh(),
    compiler_params=pltpu.CompilerParams(needs_layout_passes=False),
    scratch_types=[
        pltpu.VMEM((NPAD,), _f32), pltpu.VMEM((NPAD,), _f32),
        pltpu.VMEM((EPW,), _i32), pltpu.VMEM((EPW,), _f32),
        pltpu.VMEM((640,), _f32), pltpu.VMEM((640,), _f32),
        pltpu.VMEM_SHARED((NS, NPAD), _f32),
        pltpu.VMEM_SHARED((NS, NPAD), _f32),
    ],
)(_deg_body)


def _coef_body(po, pi, packed, ew, coefo, coefi,
               ideg_o, ideg_i, tmp, ebuf, wbuf, cob, cib):
    wid = _wid()

    for pref, idst in ((po, ideg_o), (pi, ideg_i)):
        pltpu.sync_copy(pref.at[pl.ds(0, N)], idst)
        pltpu.sync_copy(pref.at[pl.ds(NPAD, N)], tmp)

        def inv(i, _):
            v = idst[pl.ds(i * 16, 16)] + tmp[pl.ds(i * 16, 16)]
            idst[pl.ds(i * 16, 16)] = jnp.where(
                v > 0.0, 1.0 / v, jnp.zeros((16,), _f32))
            return 0

        lax.fori_loop(0, N // 16, inv, 0)

    base = wid * EPW
    pltpu.sync_copy(packed.at[pl.ds(base, EPW)], ebuf)
    pltpu.sync_copy(ew.at[pl.ds(base, EPW)], wbuf)

    def blk(i, _):
        p = ebuf[pl.ds(i * 16, 16)]
        w = wbuf[pl.ds(i * 16, 16)]
        r = p & 0xFFFF
        cc = lax.shift_right_logical(p, 16)
        cob[pl.ds(i * 16, 16)] = plsc.load_gather(ideg_o, [r]) * w
        cib[pl.ds(i * 16, 16)] = plsc.load_gather(ideg_i, [cc]) * w
        return 0

    lax.fori_loop(0, EPW // 16, blk, 0)
    pltpu.sync_copy(cob, coefo.at[pl.ds(base, EPW)])
    pltpu.sync_copy(cib, coefi.at[pl.ds(base, EPW)])


_coef_kernel = functools.partial(
    pl.kernel,
    out_type=(jax.ShapeDtypeStruct((E,), _f32),
              jax.ShapeDtypeStruct((E,), _f32)),
    mesh=_mesh(),
    compiler_params=pltpu.CompilerParams(needs_layout_passes=False),
    scratch_types=[
        pltpu.VMEM((N,), _f32), pltpu.VMEM((N,), _f32), pltpu.VMEM((N,), _f32),
        pltpu.VMEM((EPW,), _i32), pltpu.VMEM((EPW,), _f32),
        pltpu.VMEM((EPW,), _f32), pltpu.VMEM((EPW,), _f32),
    ],
)(_coef_body)


def _clen_body(Yt, packed, coefo, coefi, out,
               B0a, B0b, B1a, B1b, B2a, B2b, Sa, Sb, ebuf, cbuf):
    wid = _wid()

    def ycopy(dk, pair):
        off = (dk * W64 + 2 * wid) * N
        pltpu.sync_copy(Yt.at[pl.ds(off, N)], pair[0])
        pltpu.sync_copy(Yt.at[pl.ds(off + N, N)], pair[1])

    def ew_sub(dst, sub):
        def body(i, _):
            sl = pl.ds(i * 16, 16)
            dst[0][sl] = dst[0][sl] - sub[0][sl]
            dst[1][sl] = dst[1][sl] - sub[1][sl]
            return 0

        lax.fori_loop(0, N // 16, body, 0)

    def prop(srcb, dstb, coef_hbm, fac, src_lo):
        sa, sb = srcb
        da, db = dstb

        def chunk(ch, _):
            off = pl.multiple_of(ch * CH, 256)
            pltpu.sync_copy(packed.at[pl.ds(off, CH)], ebuf)
            pltpu.sync_copy(coef_hbm.at[pl.ds(off, CH)], cbuf)

            def blk(i, _):
                for u in range(4):
                    sl = pl.ds(i * 64 + u * 16, 16)
                    p = ebuf[sl]
                    cf = cbuf[sl]
                    lo = p & 0xFFFF
                    hi = lax.shift_right_logical(p, 16)
                    si = lo if src_lo else hi
                    di = hi if src_lo else lo
                    t = cf + cf if fac == 2.0 else cf
                    v0 = plsc.load_gather(sa, [si])
                    v1 = plsc.load_gather(sb, [si])
                    plsc.addupdate_scatter(da, [di], v0 * t)
                    plsc.addupdate_scatter(db, [di], v1 * t)
                return 0

            lax.fori_loop(0, CH // 64, blk, 0)
            return 0

        lax.fori_loop(0, NCH, chunk, 0)

    for d, (coef_hbm, src_lo) in enumerate(((coefo, True), (coefi, False))):
        bk1, bnew, free = (B0a, B0b), (B1a, B1b), (B2a, B2b)
        ycopy(d * KHOP + KHOP - 1, bk1)
        ycopy(d * KHOP + KHOP - 2, bnew)
        prop(bk1, bnew, coef_hbm, 2.0, src_lo)
        bk2, bk1 = bk1, bnew
        for k in range(KHOP - 3, 0, -1):
            bnew = free
            ycopy(d * KHOP + k, bnew)
            ew_sub(bnew, bk2)
            prop(bk1, bnew, coef_hbm, 2.0, src_lo)
            free, bk2, bk1 = bk2, bk1, bnew
        ycopy(d * KHOP, free)

        def fin(i, _):
            sl = pl.ds(i * 16, 16)
            for S_, f_, b_ in ((Sa, free[0], bk2[0]), (Sb, free[1], bk2[1])):
                v = f_[sl] - b_[sl]
                S_[sl] = v if d == 0 else S_[sl] + v
            return 0

        lax.fori_loop(0, N // 16, fin, 0)
        prop(bk1, (Sa, Sb), coef_hbm, 1.0, src_lo)

    pltpu.sync_copy(Sa, out.at[pl.ds(2 * wid * N, N)])
    pltpu.sync_copy(Sb, out.at[pl.ds((2 * wid + 1) * N, N)])


_clen_kernel = functools.partial(
    pl.kernel,
    out_type=jax.ShapeDtypeStruct((W64 * N,), _f32),
    mesh=_mesh(),
    compiler_params=pltpu.CompilerParams(needs_layout_passes=False),
    scratch_types=[
        pltpu.VMEM((N,), _f32), pltpu.VMEM((N,), _f32),
        pltpu.VMEM((N,), _f32), pltpu.VMEM((N,), _f32),
        pltpu.VMEM((N,), _f32), pltpu.VMEM((N,), _f32),
        pltpu.VMEM((N,), _f32), pltpu.VMEM((N,), _f32),
        pltpu.VMEM((CH,), _i32), pltpu.VMEM((CH,), _f32),
    ],
)(_clen_body)


def _mm_body(xr, wr, yr):
    yr[...] = jnp.dot(xr[...], wr[...], preferred_element_type=_f32).T


_mm_kernel = pl.pallas_call(
    _mm_body,
    out_shape=jax.ShapeDtypeStruct((NCOL, N), _f32),
)


def _fin_body(s_ref, b_ref, bz_ref, bh_ref, wl_ref, bl_ref, out_ref):
    sv = s_ref[...]
    z = jax.nn.sigmoid(sv[:HID, :] + bz_ref[...])
    ht = jnp.tanh(sv[HID:, :] + bh_ref[...])
    hr = jnp.maximum((1.0 - z) * ht, 0.0)
    oh = (b_ref[...] == lax.broadcasted_iota(_i32, (N, G), 1)).astype(_f32)
    sums = jnp.dot(hr, oh, preferred_element_type=_f32)
    cnts = jnp.dot(jnp.ones((1, N), _f32), oh,
                   preferred_element_type=_f32)
    pooled = sums / jnp.maximum(cnts, 1.0)
    out_ref[...] = lax.dot_general(
        pooled, wl_ref[...], (((0,), (0,)), ((), ())),
        preferred_element_type=_f32) + bl_ref[...]


_fin_kernel = pl.pallas_call(
    _fin_body,
    out_shape=jax.ShapeDtypeStruct((G, NCLS), _f32),
)


def kernel(x, edge_index, edge_weight, batch, W_z, b_z, W_r, b_r, W_h, b_h,
           W_lin, b_lin):
    row = edge_index[0].astype(_i32)
    col = edge_index[1].astype(_i32)
    packed = row | (col << 16)

    po, pi = _deg_kernel(packed, edge_weight)
    coef_o, coef_i = _coef_kernel(po, pi, packed, edge_weight)

    wcat = jnp.concatenate([W_z[:, :, :F, :], W_h[:, :, :F, :]], axis=-1)
    wflat = wcat.reshape(2 * KHOP, F, W64).transpose(1, 0, 2).reshape(F, NCOL)
    yt = _mm_kernel(x, wflat).reshape(NCOL * N)

    st = _clen_kernel(yt, packed, coef_o, coef_i).reshape(W64, N)

    return _fin_kernel(st, batch.reshape(N, 1).astype(_i32),
                       b_z.reshape(HID, 1), b_h.reshape(HID, 1),
                       W_lin, b_lin.reshape(1, NCLS))


# speedup vs baseline: 10.4053x; 1.1723x over previous
import functools

import jax
import jax.numpy as jnp
from jax import lax
from jax.experimental import pallas as pl
from jax.experimental.pallas import tpu as pltpu
from jax.experimental.pallas import tpu_sc as plsc

N = 10000
E = 320000
F = 128
HID = 32
KHOP = 5
G = 16
NCLS = 10
W64 = 2 * HID
NC, NS = 2, 16
NW = NC * NS
EPW = E // NW
NPAD = NS * 640
CH = 8000
NCH = E // CH
NCOL = 2 * KHOP * W64

_f32 = jnp.float32
_i32 = jnp.int32


def _mesh():
    return plsc.VectorSubcoreMesh(core_axis_name="c", subcore_axis_name="s",
                                  num_cores=NC, num_subcores=NS)


def _wid():
    return lax.axis_index("s") * NC + lax.axis_index("c")


def _deg_body(packed, ew, out_o, out_i, loc_o, loc_i, ebuf, wbuf, acc, tmp,
              sh_o, sh_i):
    c = lax.axis_index("c")
    s = lax.axis_index("s")
    wid = s * NC + c

    def zero(i, _):
        loc_o[pl.ds(i * 16, 16)] = jnp.zeros((16,), _f32)
        loc_i[pl.ds(i * 16, 16)] = jnp.zeros((16,), _f32)
        return 0

    lax.fori_loop(0, NPAD // 16, zero, 0)

    base = wid * EPW
    pltpu.sync_copy(packed.at[pl.ds(base, EPW)], ebuf)
    pltpu.sync_copy(ew.at[pl.ds(base, EPW)], wbuf)

    def blk(i, _):
        p = ebuf[pl.ds(i * 16, 16)]
        w = wbuf[pl.ds(i * 16, 16)]
        r = p & 0xFFFF
        cc = lax.shift_right_logical(p, 16)
        plsc.addupdate_scatter(loc_o, [r], w)
        plsc.addupdate_scatter(loc_i, [cc], w)
        return 0

    lax.fori_loop(0, EPW // 16, blk, 0)

    pltpu.sync_copy(loc_o, sh_o.at[s])
    pltpu.sync_copy(loc_i, sh_i.at[s])
    plsc.subcore_barrier()

    off = s * 640
    for sh, outref in ((sh_o, out_o), (sh_i, out_i)):
        def zero2(i, _):
            acc[pl.ds(i * 16, 16)] = jnp.zeros((16,), _f32)
            return 0

        lax.fori_loop(0, 40, zero2, 0)
        for r in range(NS):
            pltpu.sync_copy(sh.at[r, pl.ds(off, 640)], tmp)

            def addl(i, _):
                acc[pl.ds(i * 16, 16)] = (acc[pl.ds(i * 16, 16)]
                                          + tmp[pl.ds(i * 16, 16)])
                return 0

            lax.fori_loop(0, 40, addl, 0)
        pltpu.sync_copy(acc, outref.at[pl.ds(c * NPAD + off, 640)])


_deg_kernel = functools.partial(
    pl.kernel,
    out_type=(jax.ShapeDtypeStruct((NC * NPAD,), _f32),
              jax.ShapeDtypeStruct((NC * NPAD,), _f32)),
    mesh=_mesh(),
    compiler_params=pltpu.CompilerParams(needs_layout_passes=False),
    scratch_types=[
        pltpu.VMEM((NPAD,), _f32), pltpu.VMEM((NPAD,), _f32),
        pltpu.VMEM((EPW,), _i32), pltpu.VMEM((EPW,), _f32),
        pltpu.VMEM((640,), _f32), pltpu.VMEM((640,), _f32),
        pltpu.VMEM_SHARED((NS, NPAD), _f32),
        pltpu.VMEM_SHARED((NS, NPAD), _f32),
    ],
)(_deg_body)


def _coef_body(po, pi, packed, ew, coefo, coefi,
               ideg_o, ideg_i, tmp, ebuf, wbuf, cob, cib):
    wid = _wid()

    for pref, idst in ((po, ideg_o), (pi, ideg_i)):
        pltpu.sync_copy(pref.at[pl.ds(0, N)], idst)
        pltpu.sync_copy(pref.at[pl.ds(NPAD, N)], tmp)

        def inv(i, _):
            v = idst[pl.ds(i * 16, 16)] + tmp[pl.ds(i * 16, 16)]
            idst[pl.ds(i * 16, 16)] = jnp.where(
                v > 0.0, 1.0 / v, jnp.zeros((16,), _f32))
            return 0

        lax.fori_loop(0, N // 16, inv, 0)

    base = wid * EPW
    pltpu.sync_copy(packed.at[pl.ds(base, EPW)], ebuf)
    pltpu.sync_copy(ew.at[pl.ds(base, EPW)], wbuf)

    def blk(i, _):
        p = ebuf[pl.ds(i * 16, 16)]
        w = wbuf[pl.ds(i * 16, 16)]
        r = p & 0xFFFF
        cc = lax.shift_right_logical(p, 16)
        cob[pl.ds(i * 16, 16)] = plsc.load_gather(ideg_o, [r]) * w
        cib[pl.ds(i * 16, 16)] = plsc.load_gather(ideg_i, [cc]) * w
        return 0

    lax.fori_loop(0, EPW // 16, blk, 0)
    pltpu.sync_copy(cob, coefo.at[pl.ds(base, EPW)])
    pltpu.sync_copy(cib, coefi.at[pl.ds(base, EPW)])


_coef_kernel = functools.partial(
    pl.kernel,
    out_type=(jax.ShapeDtypeStruct((E,), _f32),
              jax.ShapeDtypeStruct((E,), _f32)),
    mesh=_mesh(),
    compiler_params=pltpu.CompilerParams(needs_layout_passes=False),
    scratch_types=[
        pltpu.VMEM((N,), _f32), pltpu.VMEM((N,), _f32), pltpu.VMEM((N,), _f32),
        pltpu.VMEM((EPW,), _i32), pltpu.VMEM((EPW,), _f32),
        pltpu.VMEM((EPW,), _f32), pltpu.VMEM((EPW,), _f32),
    ],
)(_coef_body)


def _clen_body(Yt, packed, coefo, coefi, out,
               B0a, B0b, B1a, B1b, B2a, B2b, Sa, Sb,
               e0, c0, e1, c1, sem0, sem1):
    wid = _wid()

    def ycopy(dk, pair):
        off = (dk * W64 + 2 * wid) * N
        pltpu.sync_copy(Yt.at[pl.ds(off, N)], pair[0])
        pltpu.sync_copy(Yt.at[pl.ds(off + N, N)], pair[1])

    def ew_sub(dst, sub):
        def body(i, _):
            sl = pl.ds(i * 16, 16)
            dst[0][sl] = dst[0][sl] - sub[0][sl]
            dst[1][sl] = dst[1][sl] - sub[1][sl]
            return 0

        lax.fori_loop(0, N // 16, body, 0)

    def prop(srcb, dstb, coef_hbm, fac, src_lo):
        sa, sb = srcb
        da, db = dstb
        bufs = ((e0, c0, sem0), (e1, c1, sem1))

        def issue(ch, eb, cb, sem):
            off = pl.multiple_of(lax.rem(ch, NCH) * CH, 64)
            pltpu.async_copy(packed.at[pl.ds(off, CH)], eb, sem)
            pltpu.async_copy(coef_hbm.at[pl.ds(off, CH)], cb, sem)

        def wait_pair(eb, cb, sem):
            pltpu.make_async_copy(packed.at[pl.ds(0, CH)], eb, sem).wait()
            pltpu.make_async_copy(coef_hbm.at[pl.ds(0, CH)], cb, sem).wait()

        def process(eb, cb):
            def blk(i, _):
                for u in range(4):
                    sl = pl.ds(i * 64 + u * 16, 16)
                    p = eb[sl]
                    cf = cb[sl]
                    lo = p & 0xFFFF
                    hi = lax.shift_right_logical(p, 16)
                    si = lo if src_lo else hi
                    di = hi if src_lo else lo
                    t = cf + cf if fac == 2.0 else cf
                    v0 = plsc.load_gather(sa, [si])
                    v1 = plsc.load_gather(sb, [si])
                    plsc.addupdate_scatter(da, [di], v0 * t)
                    plsc.addupdate_scatter(db, [di], v1 * t)
                return 0

            lax.fori_loop(0, CH // 64, blk, 0)

        issue(0, *bufs[0])
        issue(1, *bufs[1])

        def pairbody(g, _):
            for b, (eb, cb, sem) in enumerate(bufs):
                wait_pair(eb, cb, sem)
                process(eb, cb)
                issue(2 * g + b + 2, eb, cb, sem)
            return 0

        lax.fori_loop(0, NCH // 2, pairbody, 0)
        for eb, cb, sem in bufs:
            wait_pair(eb, cb, sem)

    for d, (coef_hbm, src_lo) in enumerate(((coefo, True), (coefi, False))):
        bk1, bnew, free = (B0a, B0b), (B1a, B1b), (B2a, B2b)
        ycopy(d * KHOP + KHOP - 1, bk1)
        ycopy(d * KHOP + KHOP - 2, bnew)
        prop(bk1, bnew, coef_hbm, 2.0, src_lo)
        bk2, bk1 = bk1, bnew
        for k in range(KHOP - 3, 0, -1):
            bnew = free
            ycopy(d * KHOP + k, bnew)
            ew_sub(bnew, bk2)
            prop(bk1, bnew, coef_hbm, 2.0, src_lo)
            free, bk2, bk1 = bk2, bk1, bnew
        ycopy(d * KHOP, free)

        def fin(i, _):
            sl = pl.ds(i * 16, 16)
            for S_, f_, b_ in ((Sa, free[0], bk2[0]), (Sb, free[1], bk2[1])):
                v = f_[sl] - b_[sl]
                S_[sl] = v if d == 0 else S_[sl] + v
            return 0

        lax.fori_loop(0, N // 16, fin, 0)
        prop(bk1, (Sa, Sb), coef_hbm, 1.0, src_lo)

    pltpu.sync_copy(Sa, out.at[pl.ds(2 * wid * N, N)])
    pltpu.sync_copy(Sb, out.at[pl.ds((2 * wid + 1) * N, N)])


_clen_kernel = functools.partial(
    pl.kernel,
    out_type=jax.ShapeDtypeStruct((W64 * N,), _f32),
    mesh=_mesh(),
    compiler_params=pltpu.CompilerParams(needs_layout_passes=False),
    scratch_types=[
        pltpu.VMEM((N,), _f32), pltpu.VMEM((N,), _f32),
        pltpu.VMEM((N,), _f32), pltpu.VMEM((N,), _f32),
        pltpu.VMEM((N,), _f32), pltpu.VMEM((N,), _f32),
        pltpu.VMEM((N,), _f32), pltpu.VMEM((N,), _f32),
        pltpu.VMEM((CH,), _i32), pltpu.VMEM((CH,), _f32),
        pltpu.VMEM((CH,), _i32), pltpu.VMEM((CH,), _f32),
        pltpu.SemaphoreType.DMA, pltpu.SemaphoreType.DMA,
    ],
)(_clen_body)


def _mm_body(xr, wr, yr):
    yr[...] = jnp.dot(xr[...], wr[...], preferred_element_type=_f32).T


_mm_kernel = pl.pallas_call(
    _mm_body,
    out_shape=jax.ShapeDtypeStruct((NCOL, N), _f32),
)


def _fin_body(s_ref, b_ref, bz_ref, bh_ref, wl_ref, bl_ref, out_ref):
    sv = s_ref[...]
    z = jax.nn.sigmoid(sv[:HID, :] + bz_ref[...])
    ht = jnp.tanh(sv[HID:, :] + bh_ref[...])
    hr = jnp.maximum((1.0 - z) * ht, 0.0)
    oh = (b_ref[...] == lax.broadcasted_iota(_i32, (N, G), 1)).astype(_f32)
    sums = jnp.dot(hr, oh, preferred_element_type=_f32)
    cnts = jnp.dot(jnp.ones((1, N), _f32), oh,
                   preferred_element_type=_f32)
    pooled = sums / jnp.maximum(cnts, 1.0)
    out_ref[...] = lax.dot_general(
        pooled, wl_ref[...], (((0,), (0,)), ((), ())),
        preferred_element_type=_f32) + bl_ref[...]


_fin_kernel = pl.pallas_call(
    _fin_body,
    out_shape=jax.ShapeDtypeStruct((G, NCLS), _f32),
)


def kernel(x, edge_index, edge_weight, batch, W_z, b_z, W_r, b_r, W_h, b_h,
           W_lin, b_lin):
    row = edge_index[0].astype(_i32)
    col = edge_index[1].astype(_i32)
    packed = row | (col << 16)

    po, pi = _deg_kernel(packed, edge_weight)
    coef_o, coef_i = _coef_kernel(po, pi, packed, edge_weight)

    wcat = jnp.concatenate([W_z[:, :, :F, :], W_h[:, :, :F, :]], axis=-1)
    wflat = wcat.reshape(2 * KHOP, F, W64).transpose(1, 0, 2).reshape(F, NCOL)
    yt = _mm_kernel(x, wflat).reshape(NCOL * N)

    st = _clen_kernel(yt, packed, coef_o, coef_i).reshape(W64, N)

    return _fin_kernel(st, batch.reshape(N, 1).astype(_i32),
                       b_z.reshape(HID, 1), b_h.reshape(HID, 1),
                       W_lin, b_lin.reshape(1, NCLS))


# speedup vs baseline: 23.2204x; 2.2316x over previous
import functools

import jax
import jax.numpy as jnp
from jax import lax
from jax.experimental import pallas as pl
from jax.experimental.pallas import tpu as pltpu
from jax.experimental.pallas import tpu_sc as plsc

N = 10000
E = 320000
F = 128
HID = 32
KHOP = 5
G = 16
NCLS = 10
W64 = 2 * HID
NC, NS = 2, 16
NW = NC * NS
EPW = E // NW
NPAD = NS * 640
CH = 8000
NCH = E // CH
NCOL = 2 * KHOP * W64

_f32 = jnp.float32
_i32 = jnp.int32


def _mesh():
    return plsc.VectorSubcoreMesh(core_axis_name="c", subcore_axis_name="s",
                                  num_cores=NC, num_subcores=NS)


def _wid():
    return lax.axis_index("s") * NC + lax.axis_index("c")


def _deg_body(packed, ew, out_o, out_i, loc_o, loc_i, ebuf, wbuf, acc, tmp,
              sh_o, sh_i):
    c = lax.axis_index("c")
    s = lax.axis_index("s")
    wid = s * NC + c

    def zero(i, _):
        loc_o[pl.ds(i * 16, 16)] = jnp.zeros((16,), _f32)
        loc_i[pl.ds(i * 16, 16)] = jnp.zeros((16,), _f32)
        return 0

    lax.fori_loop(0, NPAD // 16, zero, 0)

    base = wid * EPW
    pltpu.sync_copy(packed.at[pl.ds(base, EPW)], ebuf)
    pltpu.sync_copy(ew.at[pl.ds(base, EPW)], wbuf)

    def blk(i, _):
        p = ebuf[pl.ds(i * 16, 16)]
        w = wbuf[pl.ds(i * 16, 16)]
        r = p & 0xFFFF
        cc = lax.shift_right_logical(p, 16)
        plsc.addupdate_scatter(loc_o, [r], w)
        plsc.addupdate_scatter(loc_i, [cc], w)
        return 0

    lax.fori_loop(0, EPW // 16, blk, 0)

    pltpu.sync_copy(loc_o, sh_o.at[s])
    pltpu.sync_copy(loc_i, sh_i.at[s])
    plsc.subcore_barrier()

    off = s * 640
    for sh, outref in ((sh_o, out_o), (sh_i, out_i)):
        def zero2(i, _):
            acc[pl.ds(i * 16, 16)] = jnp.zeros((16,), _f32)
            return 0

        lax.fori_loop(0, 40, zero2, 0)
        for r in range(NS):
            pltpu.sync_copy(sh.at[r, pl.ds(off, 640)], tmp)

            def addl(i, _):
                acc[pl.ds(i * 16, 16)] = (acc[pl.ds(i * 16, 16)]
                                          + tmp[pl.ds(i * 16, 16)])
                return 0

            lax.fori_loop(0, 40, addl, 0)
        pltpu.sync_copy(acc, outref.at[pl.ds(c * NPAD + off, 640)])


_deg_kernel = functools.partial(
    pl.kernel,
    out_type=(jax.ShapeDtypeStruct((NC * NPAD,), _f32),
              jax.ShapeDtypeStruct((NC * NPAD,), _f32)),
    mesh=_mesh(),
    compiler_params=pltpu.CompilerParams(needs_layout_passes=False),
    scratch_types=[
        pltpu.VMEM((NPAD,), _f32), pltpu.VMEM((NPAD,), _f32),
        pltpu.VMEM((EPW,), _i32), pltpu.VMEM((EPW,), _f32),
        pltpu.VMEM((640,), _f32), pltpu.VMEM((640,), _f32),
        pltpu.VMEM_SHARED((NS, NPAD), _f32),
        pltpu.VMEM_SHARED((NS, NPAD), _f32),
    ],
)(_deg_body)


def _coef_body(po, pi, packed, ew, coefo, coefi,
               ideg_o, ideg_i, tmp, ebuf, wbuf, cob, cib):
    wid = _wid()

    for pref, idst in ((po, ideg_o), (pi, ideg_i)):
        pltpu.sync_copy(pref.at[pl.ds(0, N)], idst)
        pltpu.sync_copy(pref.at[pl.ds(NPAD, N)], tmp)

        def inv(i, _):
            v = idst[pl.ds(i * 16, 16)] + tmp[pl.ds(i * 16, 16)]
            idst[pl.ds(i * 16, 16)] = jnp.where(
                v > 0.0, 1.0 / v, jnp.zeros((16,), _f32))
            return 0

        lax.fori_loop(0, N // 16, inv, 0)

    base = wid * EPW
    pltpu.sync_copy(packed.at[pl.ds(base, EPW)], ebuf)
    pltpu.sync_copy(ew.at[pl.ds(base, EPW)], wbuf)

    def blk(i, _):
        p = ebuf[pl.ds(i * 16, 16)]
        w = wbuf[pl.ds(i * 16, 16)]
        r = p & 0xFFFF
        cc = lax.shift_right_logical(p, 16)
        cob[pl.ds(i * 16, 16)] = plsc.load_gather(ideg_o, [r]) * w
        cib[pl.ds(i * 16, 16)] = plsc.load_gather(ideg_i, [cc]) * w
        return 0

    lax.fori_loop(0, EPW // 16, blk, 0)
    pltpu.sync_copy(cob, coefo.at[pl.ds(base, EPW)])
    pltpu.sync_copy(cib, coefi.at[pl.ds(base, EPW)])


_coef_kernel = functools.partial(
    pl.kernel,
    out_type=(jax.ShapeDtypeStruct((E,), _f32),
              jax.ShapeDtypeStruct((E,), _f32)),
    mesh=_mesh(),
    compiler_params=pltpu.CompilerParams(needs_layout_passes=False),
    scratch_types=[
        pltpu.VMEM((N,), _f32), pltpu.VMEM((N,), _f32), pltpu.VMEM((N,), _f32),
        pltpu.VMEM((EPW,), _i32), pltpu.VMEM((EPW,), _f32),
        pltpu.VMEM((EPW,), _f32), pltpu.VMEM((EPW,), _f32),
    ],
)(_coef_body)


def _clen_body(Yt, packed, coefo, coefi, out,
               B0a, B0b, B1a, B1b, B2a, B2b, Sa, Sb,
               e0, c0, e1, c1, sem0, sem1):
    wid = _wid()

    def ycopy(dk, pair):
        off = (dk * W64 + 2 * wid) * N
        pltpu.sync_copy(Yt.at[pl.ds(off, N)], pair[0])
        pltpu.sync_copy(Yt.at[pl.ds(off + N, N)], pair[1])

    def ew_sub(dst, sub):
        def body(i, _):
            sl = pl.ds(i * 16, 16)
            dst[0][sl] = dst[0][sl] - sub[0][sl]
            dst[1][sl] = dst[1][sl] - sub[1][sl]
            return 0

        lax.fori_loop(0, N // 16, body, 0)

    def prop(srcb, dstb, coef_hbm, fac, src_lo):
        sa, sb = srcb
        da, db = dstb
        bufs = ((e0, c0, sem0), (e1, c1, sem1))

        def issue(ch, eb, cb, sem):
            off = pl.multiple_of(lax.rem(ch, NCH) * CH, 64)
            pltpu.async_copy(packed.at[pl.ds(off, CH)], eb, sem)
            pltpu.async_copy(coef_hbm.at[pl.ds(off, CH)], cb, sem)

        def wait_pair(eb, cb, sem):
            pltpu.make_async_copy(packed.at[pl.ds(0, CH)], eb, sem).wait()
            pltpu.make_async_copy(coef_hbm.at[pl.ds(0, CH)], cb, sem).wait()

        def process(eb, cb):
            @plsc.parallel_loop(0, CH // 16, unroll=8)
            def blk(i):
                sl = pl.ds(i * 16, 16)
                p = eb[sl]
                cf = cb[sl]
                lo = p & 0xFFFF
                hi = lax.shift_right_logical(p, 16)
                si = lo if src_lo else hi
                di = hi if src_lo else lo
                t = cf + cf if fac == 2.0 else cf
                v0 = plsc.load_gather(sa, [si])
                v1 = plsc.load_gather(sb, [si])
                plsc.addupdate_scatter(da, [di], v0 * t)
                plsc.addupdate_scatter(db, [di], v1 * t)

        issue(0, *bufs[0])
        issue(1, *bufs[1])

        def pairbody(g, _):
            for b, (eb, cb, sem) in enumerate(bufs):
                wait_pair(eb, cb, sem)
                process(eb, cb)
                issue(2 * g + b + 2, eb, cb, sem)
            return 0

        lax.fori_loop(0, NCH // 2, pairbody, 0)
        for eb, cb, sem in bufs:
            wait_pair(eb, cb, sem)

    for d, (coef_hbm, src_lo) in enumerate(((coefo, True), (coefi, False))):
        bk1, bnew, free = (B0a, B0b), (B1a, B1b), (B2a, B2b)
        ycopy(d * KHOP + KHOP - 1, bk1)
        ycopy(d * KHOP + KHOP - 2, bnew)
        prop(bk1, bnew, coef_hbm, 2.0, src_lo)
        bk2, bk1 = bk1, bnew
        for k in range(KHOP - 3, 0, -1):
            bnew = free
            ycopy(d * KHOP + k, bnew)
            ew_sub(bnew, bk2)
            prop(bk1, bnew, coef_hbm, 2.0, src_lo)
            free, bk2, bk1 = bk2, bk1, bnew
        ycopy(d * KHOP, free)

        def fin(i, _):
            sl = pl.ds(i * 16, 16)
            for S_, f_, b_ in ((Sa, free[0], bk2[0]), (Sb, free[1], bk2[1])):
                v = f_[sl] - b_[sl]
                S_[sl] = v if d == 0 else S_[sl] + v
            return 0

        lax.fori_loop(0, N // 16, fin, 0)
        prop(bk1, (Sa, Sb), coef_hbm, 1.0, src_lo)

    pltpu.sync_copy(Sa, out.at[pl.ds(2 * wid * N, N)])
    pltpu.sync_copy(Sb, out.at[pl.ds((2 * wid + 1) * N, N)])


_clen_kernel = functools.partial(
    pl.kernel,
    out_type=jax.ShapeDtypeStruct((W64 * N,), _f32),
    mesh=_mesh(),
    compiler_params=pltpu.CompilerParams(needs_layout_passes=False),
    scratch_types=[
        pltpu.VMEM((N,), _f32), pltpu.VMEM((N,), _f32),
        pltpu.VMEM((N,), _f32), pltpu.VMEM((N,), _f32),
        pltpu.VMEM((N,), _f32), pltpu.VMEM((N,), _f32),
        pltpu.VMEM((N,), _f32), pltpu.VMEM((N,), _f32),
        pltpu.VMEM((CH,), _i32), pltpu.VMEM((CH,), _f32),
        pltpu.VMEM((CH,), _i32), pltpu.VMEM((CH,), _f32),
        pltpu.SemaphoreType.DMA, pltpu.SemaphoreType.DMA,
    ],
)(_clen_body)


def _mm_body(xr, wr, yr):
    yr[...] = jnp.dot(xr[...], wr[...], preferred_element_type=_f32).T


_mm_kernel = pl.pallas_call(
    _mm_body,
    out_shape=jax.ShapeDtypeStruct((NCOL, N), _f32),
)


def _fin_body(s_ref, b_ref, bz_ref, bh_ref, wl_ref, bl_ref, out_ref):
    sv = s_ref[...]
    z = jax.nn.sigmoid(sv[:HID, :] + bz_ref[...])
    ht = jnp.tanh(sv[HID:, :] + bh_ref[...])
    hr = jnp.maximum((1.0 - z) * ht, 0.0)
    oh = (b_ref[...] == lax.broadcasted_iota(_i32, (N, G), 1)).astype(_f32)
    sums = jnp.dot(hr, oh, preferred_element_type=_f32)
    cnts = jnp.dot(jnp.ones((1, N), _f32), oh,
                   preferred_element_type=_f32)
    pooled = sums / jnp.maximum(cnts, 1.0)
    out_ref[...] = lax.dot_general(
        pooled, wl_ref[...], (((0,), (0,)), ((), ())),
        preferred_element_type=_f32) + bl_ref[...]


_fin_kernel = pl.pallas_call(
    _fin_body,
    out_shape=jax.ShapeDtypeStruct((G, NCLS), _f32),
)


def kernel(x, edge_index, edge_weight, batch, W_z, b_z, W_r, b_r, W_h, b_h,
           W_lin, b_lin):
    row = edge_index[0].astype(_i32)
    col = edge_index[1].astype(_i32)
    packed = row | (col << 16)

    po, pi = _deg_kernel(packed, edge_weight)
    coef_o, coef_i = _coef_kernel(po, pi, packed, edge_weight)

    wcat = jnp.concatenate([W_z[:, :, :F, :], W_h[:, :, :F, :]], axis=-1)
    wflat = wcat.reshape(2 * KHOP, F, W64).transpose(1, 0, 2).reshape(F, NCOL)
    yt = _mm_kernel(x, wflat).reshape(NCOL * N)

    st = _clen_kernel(yt, packed, coef_o, coef_i).reshape(W64, N)

    return _fin_kernel(st, batch.reshape(N, 1).astype(_i32),
                       b_z.reshape(HID, 1), b_h.reshape(HID, 1),
                       W_lin, b_lin.reshape(1, NCLS))


# speedup vs baseline: 23.5554x; 1.0144x over previous
import functools

import jax
import jax.numpy as jnp
from jax import lax
from jax.experimental import pallas as pl
from jax.experimental.pallas import tpu as pltpu
from jax.experimental.pallas import tpu_sc as plsc

N = 10000
E = 320000
F = 128
HID = 32
KHOP = 5
G = 16
NCLS = 10
W64 = 2 * HID
NC, NS = 2, 16
NW = NC * NS
EPW = E // NW
NPAD = NS * 640
CH = 8000
NCH = E // CH
NCOL = 2 * KHOP * W64

_f32 = jnp.float32
_i32 = jnp.int32


def _mesh():
    return plsc.VectorSubcoreMesh(core_axis_name="c", subcore_axis_name="s",
                                  num_cores=NC, num_subcores=NS)


def _wid():
    return lax.axis_index("s") * NC + lax.axis_index("c")


def _deg_body(packed, ew, out_o, out_i, loc_o, loc_i, ebuf, wbuf, acc, tmp,
              sh_o, sh_i):
    c = lax.axis_index("c")
    s = lax.axis_index("s")
    wid = s * NC + c

    @plsc.parallel_loop(0, NPAD // 16, unroll=4)
    def zero(i):
        loc_o[pl.ds(i * 16, 16)] = jnp.zeros((16,), _f32)
        loc_i[pl.ds(i * 16, 16)] = jnp.zeros((16,), _f32)

    base = wid * EPW
    pltpu.sync_copy(packed.at[pl.ds(base, EPW)], ebuf)
    pltpu.sync_copy(ew.at[pl.ds(base, EPW)], wbuf)

    @plsc.parallel_loop(0, EPW // 16, unroll=8)
    def blk(i):
        p = ebuf[pl.ds(i * 16, 16)]
        w = wbuf[pl.ds(i * 16, 16)]
        r = p & 0xFFFF
        cc = lax.shift_right_logical(p, 16)
        plsc.addupdate_scatter(loc_o, [r], w)
        plsc.addupdate_scatter(loc_i, [cc], w)

    pltpu.sync_copy(loc_o, sh_o.at[s])
    pltpu.sync_copy(loc_i, sh_i.at[s])
    plsc.subcore_barrier()

    off = s * 640
    for sh, outref in ((sh_o, out_o), (sh_i, out_i)):
        @plsc.parallel_loop(0, 40, unroll=4)
        def zero2(i):
            acc[pl.ds(i * 16, 16)] = jnp.zeros((16,), _f32)
        for r in range(NS):
            pltpu.sync_copy(sh.at[r, pl.ds(off, 640)], tmp)

            @plsc.parallel_loop(0, 40, unroll=4)
            def addl(i):
                acc[pl.ds(i * 16, 16)] = (acc[pl.ds(i * 16, 16)]
                                          + tmp[pl.ds(i * 16, 16)])
        pltpu.sync_copy(acc, outref.at[pl.ds(c * NPAD + off, 640)])


_deg_kernel = functools.partial(
    pl.kernel,
    out_type=(jax.ShapeDtypeStruct((NC * NPAD,), _f32),
              jax.ShapeDtypeStruct((NC * NPAD,), _f32)),
    mesh=_mesh(),
    compiler_params=pltpu.CompilerParams(needs_layout_passes=False),
    scratch_types=[
        pltpu.VMEM((NPAD,), _f32), pltpu.VMEM((NPAD,), _f32),
        pltpu.VMEM((EPW,), _i32), pltpu.VMEM((EPW,), _f32),
        pltpu.VMEM((640,), _f32), pltpu.VMEM((640,), _f32),
        pltpu.VMEM_SHARED((NS, NPAD), _f32),
        pltpu.VMEM_SHARED((NS, NPAD), _f32),
    ],
)(_deg_body)


def _coef_body(po, pi, packed, ew, coefo, coefi,
               ideg_o, ideg_i, tmp, ebuf, wbuf, cob, cib):
    wid = _wid()

    for pref, idst in ((po, ideg_o), (pi, ideg_i)):
        pltpu.sync_copy(pref.at[pl.ds(0, N)], idst)
        pltpu.sync_copy(pref.at[pl.ds(NPAD, N)], tmp)

        @plsc.parallel_loop(0, N // 16, unroll=4)
        def inv(i):
            v = idst[pl.ds(i * 16, 16)] + tmp[pl.ds(i * 16, 16)]
            idst[pl.ds(i * 16, 16)] = jnp.where(
                v > 0.0, 1.0 / v, jnp.zeros((16,), _f32))

    base = wid * EPW
    pltpu.sync_copy(packed.at[pl.ds(base, EPW)], ebuf)
    pltpu.sync_copy(ew.at[pl.ds(base, EPW)], wbuf)

    @plsc.parallel_loop(0, EPW // 16, unroll=8)
    def blk(i):
        p = ebuf[pl.ds(i * 16, 16)]
        w = wbuf[pl.ds(i * 16, 16)]
        r = p & 0xFFFF
        cc = lax.shift_right_logical(p, 16)
        cob[pl.ds(i * 16, 16)] = plsc.load_gather(ideg_o, [r]) * w
        cib[pl.ds(i * 16, 16)] = plsc.load_gather(ideg_i, [cc]) * w
    pltpu.sync_copy(cob, coefo.at[pl.ds(base, EPW)])
    pltpu.sync_copy(cib, coefi.at[pl.ds(base, EPW)])


_coef_kernel = functools.partial(
    pl.kernel,
    out_type=(jax.ShapeDtypeStruct((E,), _f32),
              jax.ShapeDtypeStruct((E,), _f32)),
    mesh=_mesh(),
    compiler_params=pltpu.CompilerParams(needs_layout_passes=False),
    scratch_types=[
        pltpu.VMEM((N,), _f32), pltpu.VMEM((N,), _f32), pltpu.VMEM((N,), _f32),
        pltpu.VMEM((EPW,), _i32), pltpu.VMEM((EPW,), _f32),
        pltpu.VMEM((EPW,), _f32), pltpu.VMEM((EPW,), _f32),
    ],
)(_coef_body)


def _clen_body(Yt, packed, coefo, coefi, out,
               B0a, B0b, B1a, B1b, B2a, B2b, Sa, Sb,
               e0, c0, e1, c1, sem0, sem1):
    wid = _wid()

    def ycopy(dk, pair):
        off = (dk * W64 + 2 * wid) * N
        pltpu.sync_copy(Yt.at[pl.ds(off, N)], pair[0])
        pltpu.sync_copy(Yt.at[pl.ds(off + N, N)], pair[1])

    def ew_sub(dst, sub):
        @plsc.parallel_loop(0, N // 16, unroll=4)
        def body(i):
            sl = pl.ds(i * 16, 16)
            dst[0][sl] = dst[0][sl] - sub[0][sl]
            dst[1][sl] = dst[1][sl] - sub[1][sl]

    def prop(srcb, dstb, coef_hbm, fac, src_lo):
        sa, sb = srcb
        da, db = dstb
        bufs = ((e0, c0, sem0), (e1, c1, sem1))

        def issue(ch, eb, cb, sem):
            off = pl.multiple_of(lax.rem(ch, NCH) * CH, 64)
            pltpu.async_copy(packed.at[pl.ds(off, CH)], eb, sem)
            pltpu.async_copy(coef_hbm.at[pl.ds(off, CH)], cb, sem)

        def wait_pair(eb, cb, sem):
            pltpu.make_async_copy(packed.at[pl.ds(0, CH)], eb, sem).wait()
            pltpu.make_async_copy(coef_hbm.at[pl.ds(0, CH)], cb, sem).wait()

        def process(eb, cb):
            @plsc.parallel_loop(0, CH // 16, unroll=16)
            def blk(i):
                sl = pl.ds(i * 16, 16)
                p = eb[sl]
                cf = cb[sl]
                lo = p & 0xFFFF
                hi = lax.shift_right_logical(p, 16)
                si = lo if src_lo else hi
                di = hi if src_lo else lo
                t = cf + cf if fac == 2.0 else cf
                v0 = plsc.load_gather(sa, [si])
                v1 = plsc.load_gather(sb, [si])
                plsc.addupdate_scatter(da, [di], v0 * t)
                plsc.addupdate_scatter(db, [di], v1 * t)

        issue(0, *bufs[0])
        issue(1, *bufs[1])

        def pairbody(g, _):
            for b, (eb, cb, sem) in enumerate(bufs):
                wait_pair(eb, cb, sem)
                process(eb, cb)
                issue(2 * g + b + 2, eb, cb, sem)
            return 0

        lax.fori_loop(0, NCH // 2, pairbody, 0)
        for eb, cb, sem in bufs:
            wait_pair(eb, cb, sem)

    for d, (coef_hbm, src_lo) in enumerate(((coefo, True), (coefi, False))):
        bk1, bnew, free = (B0a, B0b), (B1a, B1b), (B2a, B2b)
        ycopy(d * KHOP + KHOP - 1, bk1)
        ycopy(d * KHOP + KHOP - 2, bnew)
        prop(bk1, bnew, coef_hbm, 2.0, src_lo)
        bk2, bk1 = bk1, bnew
        for k in range(KHOP - 3, 0, -1):
            bnew = free
            ycopy(d * KHOP + k, bnew)
            ew_sub(bnew, bk2)
            prop(bk1, bnew, coef_hbm, 2.0, src_lo)
            free, bk2, bk1 = bk2, bk1, bnew
        ycopy(d * KHOP, free)

        @plsc.parallel_loop(0, N // 16, unroll=4)
        def fin(i):
            sl = pl.ds(i * 16, 16)
            for S_, f_, b_ in ((Sa, free[0], bk2[0]), (Sb, free[1], bk2[1])):
                v = f_[sl] - b_[sl]
                S_[sl] = v if d == 0 else S_[sl] + v
        prop(bk1, (Sa, Sb), coef_hbm, 1.0, src_lo)

    pltpu.sync_copy(Sa, out.at[pl.ds(2 * wid * N, N)])
    pltpu.sync_copy(Sb, out.at[pl.ds((2 * wid + 1) * N, N)])


_clen_kernel = functools.partial(
    pl.kernel,
    out_type=jax.ShapeDtypeStruct((W64 * N,), _f32),
    mesh=_mesh(),
    compiler_params=pltpu.CompilerParams(needs_layout_passes=False),
    scratch_types=[
        pltpu.VMEM((N,), _f32), pltpu.VMEM((N,), _f32),
        pltpu.VMEM((N,), _f32), pltpu.VMEM((N,), _f32),
        pltpu.VMEM((N,), _f32), pltpu.VMEM((N,), _f32),
        pltpu.VMEM((N,), _f32), pltpu.VMEM((N,), _f32),
        pltpu.VMEM((CH,), _i32), pltpu.VMEM((CH,), _f32),
        pltpu.VMEM((CH,), _i32), pltpu.VMEM((CH,), _f32),
        pltpu.SemaphoreType.DMA, pltpu.SemaphoreType.DMA,
    ],
)(_clen_body)


def _mm_body(xr, wr, yr):
    yr[...] = jnp.dot(xr[...], wr[...], preferred_element_type=_f32).T


_mm_kernel = pl.pallas_call(
    _mm_body,
    out_shape=jax.ShapeDtypeStruct((NCOL, N), _f32),
)


def _fin_body(s_ref, b_ref, bz_ref, bh_ref, wl_ref, bl_ref, out_ref):
    sv = s_ref[...]
    z = jax.nn.sigmoid(sv[:HID, :] + bz_ref[...])
    ht = jnp.tanh(sv[HID:, :] + bh_ref[...])
    hr = jnp.maximum((1.0 - z) * ht, 0.0)
    oh = (b_ref[...] == lax.broadcasted_iota(_i32, (N, G), 1)).astype(_f32)
    sums = jnp.dot(hr, oh, preferred_element_type=_f32)
    cnts = jnp.dot(jnp.ones((1, N), _f32), oh,
                   preferred_element_type=_f32)
    pooled = sums / jnp.maximum(cnts, 1.0)
    out_ref[...] = lax.dot_general(
        pooled, wl_ref[...], (((0,), (0,)), ((), ())),
        preferred_element_type=_f32) + bl_ref[...]


_fin_kernel = pl.pallas_call(
    _fin_body,
    out_shape=jax.ShapeDtypeStruct((G, NCLS), _f32),
)


def kernel(x, edge_index, edge_weight, batch, W_z, b_z, W_r, b_r, W_h, b_h,
           W_lin, b_lin):
    row = edge_index[0].astype(_i32)
    col = edge_index[1].astype(_i32)
    packed = row | (col << 16)

    po, pi = _deg_kernel(packed, edge_weight)
    coef_o, coef_i = _coef_kernel(po, pi, packed, edge_weight)

    wcat = jnp.concatenate([W_z[:, :, :F, :], W_h[:, :, :F, :]], axis=-1)
    wflat = wcat.reshape(2 * KHOP, F, W64).transpose(1, 0, 2).reshape(F, NCOL)
    yt = _mm_kernel(x, wflat).reshape(NCOL * N)

    st = _clen_kernel(yt, packed, coef_o, coef_i).reshape(W64, N)

    return _fin_kernel(st, batch.reshape(N, 1).astype(_i32),
                       b_z.reshape(HID, 1), b_h.reshape(HID, 1),
                       W_lin, b_lin.reshape(1, NCLS))
